# baseline (speedup 1.0000x reference)
import functools

import jax
import jax.numpy as jnp
from jax import lax
from jax.experimental import pallas as pl
from jax.experimental.pallas import tpu as pltpu
from jax.experimental.pallas import tpu_sc as plsc

_NUM_CORES = 2
_NUM_SUBCORES = 16
_NW = _NUM_CORES * _NUM_SUBCORES
_CHUNK = 512


@functools.partial(jax.jit, static_argnames=())
def _gather_rows(idx, table):
    n = idx.shape[0]
    d = table.shape[1]
    assert n % (_NW * _CHUNK) == 0
    per_w = n // _NW
    n_steps = per_w // _CHUNK

    mesh = plsc.VectorSubcoreMesh(core_axis_name="c", subcore_axis_name="s")

    @functools.partial(
        pl.kernel,
        mesh=mesh,
        compiler_params=pltpu.CompilerParams(use_tc_tiling_on_sc=False),
        out_type=jax.ShapeDtypeStruct((n, d), jnp.float32),
        scratch_types=[
            pltpu.VMEM((_CHUNK,), jnp.int32),
            pltpu.VMEM((_CHUNK, d), jnp.float32),
            pltpu.SemaphoreType.DMA,
        ],
    )
    def k(idx_hbm, table_hbm, out_hbm, idx_v, rows_v, sem):
        wid = lax.axis_index("s") * _NUM_CORES + lax.axis_index("c")
        base = wid * per_w

        def body(i, carry):
            off = base + i * _CHUNK
            pltpu.sync_copy(idx_hbm.at[pl.ds(off, _CHUNK)], idx_v)
            pltpu.async_copy(table_hbm.at[idx_v], rows_v, sem).wait()
            pltpu.sync_copy(rows_v, out_hbm.at[pl.ds(off, _CHUNK)])
            return carry

        lax.fori_loop(0, n_steps, body, 0)

    return k(idx, table)


def kernel(x, table):
    idx = x.reshape(-1).astype(jnp.int32)
    out = _gather_rows(idx, table)
    return out.reshape(x.shape + (table.shape[1],))


# speedup vs baseline: 1.0714x; 1.0714x over previous
import functools

import jax
import jax.numpy as jnp
from jax import lax
from jax.experimental import pallas as pl
from jax.experimental.pallas import tpu as pltpu
from jax.experimental.pallas import tpu_sc as plsc

_NUM_CORES = 2
_NUM_SUBCORES = 16
_NW = _NUM_CORES * _NUM_SUBCORES
_CHUNK = 512
_NBUF = 2


def _gather_rows(idx, table):
    n = idx.shape[0]
    d = table.shape[1]
    assert n % (_NW * _CHUNK * _NBUF) == 0
    per_w = n // _NW
    n_steps = per_w // _CHUNK

    mesh = plsc.VectorSubcoreMesh(core_axis_name="c", subcore_axis_name="s")

    scratch = (
        [pltpu.VMEM((_CHUNK,), jnp.int32) for _ in range(_NBUF)]
        + [pltpu.VMEM((_CHUNK, d), jnp.float32) for _ in range(_NBUF)]
        + [pltpu.SemaphoreType.DMA for _ in range(_NBUF)]
        + [pltpu.SemaphoreType.DMA for _ in range(_NBUF)]
        + [pltpu.SemaphoreType.DMA for _ in range(_NBUF)]
    )

    @functools.partial(
        pl.kernel,
        mesh=mesh,
        compiler_params=pltpu.CompilerParams(use_tc_tiling_on_sc=False),
        out_type=jax.ShapeDtypeStruct((n, d), jnp.float32),
        scratch_types=scratch,
    )
    def k(idx_hbm, table_hbm, out_hbm, *refs):
        idx_v = refs[0:_NBUF]
        rows_v = refs[_NBUF:2 * _NBUF]
        sem_i = refs[2 * _NBUF:3 * _NBUF]
        sem_g = refs[3 * _NBUF:4 * _NBUF]
        sem_s = refs[4 * _NBUF:5 * _NBUF]

        wid = lax.axis_index("s") * _NUM_CORES + lax.axis_index("c")
        base = wid * per_w

        for b in range(_NBUF):
            pltpu.async_copy(
                idx_hbm.at[pl.ds(base + b * _CHUNK, _CHUNK)], idx_v[b], sem_i[b]
            )

        def outer(o, carry):
            g0 = o * _NBUF
            for b in range(_NBUF):
                g = g0 + b
                off = base + g * _CHUNK
                pltpu.make_async_copy(
                    idx_hbm.at[pl.ds(off, _CHUNK)], idx_v[b], sem_i[b]
                ).wait()

                @pl.when(g >= _NBUF)
                def _():
                    pltpu.make_async_copy(
                        rows_v[b], out_hbm.at[pl.ds(off, _CHUNK)], sem_s[b]
                    ).wait()

                pltpu.async_copy(table_hbm.at[idx_v[b]], rows_v[b], sem_g[b]).wait()

                @pl.when(g + _NBUF < n_steps)
                def _():
                    pltpu.async_copy(
                        idx_hbm.at[pl.ds(off + _NBUF * _CHUNK, _CHUNK)],
                        idx_v[b],
                        sem_i[b],
                    )

                pltpu.async_copy(rows_v[b], out_hbm.at[pl.ds(off, _CHUNK)], sem_s[b])
            return carry

        lax.fori_loop(0, n_steps // _NBUF, outer, 0)

        for b in range(_NBUF):
            off = base + (n_steps - _NBUF + b) * _CHUNK
            pltpu.make_async_copy(
                rows_v[b], out_hbm.at[pl.ds(off, _CHUNK)], sem_s[b]
            ).wait()

    return k(idx, table)


def kernel(x, table):
    idx = x.reshape(-1).astype(jnp.int32)
    out = _gather_rows(idx, table)
    return out.reshape(x.shape + (table.shape[1],))
